# baseline (speedup 1.0000x reference)
import functools

import jax
import jax.numpy as jnp
from jax import lax
from jax.experimental import pallas as pl
from jax.experimental.pallas import tpu as pltpu
from jax.experimental.pallas import tpu_sc as plsc

DIM = 1024
NC, NS, L = 2, 16, 16
NW = NC * NS
R = 32


def _sc_body(x_hbm, t_hbm, emb_hbm, out_hbm,
             embv, tv, idxbuf, rkbuf, pbuf, sidx2, spm,
             xb0, xb1, si0, si1, so0, so1):
    cid = lax.axis_index("c")
    sid = lax.axis_index("s")
    wid = sid * NC + cid
    rows_per_w = x_hbm.shape[0] // NW
    nchunks = rows_per_w // R
    ngrp = rows_per_w // L
    base = wid * rows_per_w

    pltpu.sync_copy(emb_hbm, embv)
    pltpu.sync_copy(t_hbm.at[pl.ds(base, rows_per_w)], tv)

    lanes = jax.lax.iota(jnp.int32, L)

    def idx_body(i, carry):
        idxbuf[i // (R // L), pl.ds((i % (R // L)) * L, L)] = (
            lanes + base + i * L)
        return carry

    lax.fori_loop(0, ngrp, idx_body, 0)

    def _cumsum16(v):
        s = v
        for dd in (1, 2, 4, 8):
            idx = jnp.maximum(lanes - dd, 0)
            gate = jnp.minimum(jnp.maximum(lanes - (dd - 1), 0), 1)
            s = s + s.at[idx].get(mode="promise_in_bounds") * gate
        return s

    def grp_body(i, carry):
        p0, p1, p2 = carry
        tt = tv[pl.ds(i * L, L)]
        m2 = jnp.maximum(tt - 1, 0)
        m1 = tt - 2 * m2
        m0 = 1 - m1 - m2
        r0 = _cumsum16(m0)
        r1 = _cumsum16(m1)
        r2 = _cumsum16(m2)
        rk = m0 * (r0 + (p0 - 1)) + m1 * (r1 + (p1 - 1)) + m2 * (r2 + (p2 - 1))
        rkbuf[pl.ds(i * L, L)] = rk
        return (p0 + r0[L - 1], p1 + r1[L - 1], p2 + r2[L - 1])

    z = jnp.int32(0)
    n0, n1, _n2 = lax.fori_loop(0, ngrp, grp_body, (z, z, z))

    def grp2_body(i, carry):
        tt = tv[pl.ds(i * L, L)]
        m2 = jnp.maximum(tt - 1, 0)
        m1 = tt - 2 * m2
        p16 = (rkbuf[pl.ds(i * L, L)] + m1 * n0 + m2 * (n0 + n1)
               + sid * rows_per_w)
        pbuf[i // (R // L), pl.ds((i % (R // L)) * L, L)] = p16
        return carry

    lax.fori_loop(0, ngrp, grp2_body, 0)

    def inv_body(g, carry):
        pltpu.sync_copy(idxbuf.at[g], spm.at[pbuf.at[g]])
        return carry

    lax.fori_loop(0, nchunks, inv_body, 0)

    def pull_body(g, carry):
        pltpu.sync_copy(
            spm.at[pl.ds(sid * rows_per_w + g * R, R)], sidx2.at[g])
        return carry

    lax.fori_loop(0, nchunks, pull_body, 0)

    bufs, si, so = [xb0, xb1], [si0, si1], [so0, so1]
    in_d, out_d = {}, {}

    def start_in(g):
        in_d[g] = pltpu.async_copy(
            x_hbm.at[sidx2.at[g]], bufs[g & 1], si[g & 1])

    def start_out(g):
        out_d[g] = pltpu.async_copy(
            bufs[g & 1], out_hbm.at[sidx2.at[g]], so[g & 1])

    b1, b2 = n0, n0 + n1
    start_in(0)
    for g in range(nchunks):
        buf = bufs[g & 1]
        if g + 1 < nchunks:
            if g >= 1:
                out_d[g - 1].wait()
            start_in(g + 1)
        in_d[g].wait()

        g0 = g * R

        def seg_body(k, carry, buf=buf, g0=g0):
            m2k = jnp.maximum(k - 1, 0)
            m1k = k - 2 * m2k
            blo = b1 * m1k + b2 * m2k
            bhi = (b1 * (1 - m1k - m2k) + b2 * m1k
                   + jnp.int32(rows_per_w) * m2k)
            lo = jnp.clip(blo, g0, g0 + R) - g0
            hi = jnp.clip(bhi, g0, g0 + R) - g0

            def c_body(c, carry2):
                cc = c * L
                ek = embv[k, pl.ds(cc, L)]

                @plsc.parallel_loop(lo, hi, step=1, unroll=4)
                def row_body(r):
                    buf[r, pl.ds(cc, L)] = buf[r, pl.ds(cc, L)] + ek

                return carry2

            return lax.fori_loop(0, DIM // L, c_body, carry)

        lax.fori_loop(0, 3, seg_body, 0)
        start_out(g)

    out_d[nchunks - 2].wait()
    out_d[nchunks - 1].wait()


def _sc_call(x2, t1, emb):
    rows = x2.shape[0]
    rpw = rows // NW
    return pl.kernel(
        _sc_body,
        out_type=jax.ShapeDtypeStruct((rows, DIM), jnp.float32),
        mesh=plsc.VectorSubcoreMesh(core_axis_name="c", subcore_axis_name="s"),
        scratch_types=[
            pltpu.VMEM((3, DIM), jnp.float32),
            pltpu.VMEM((rpw,), jnp.int32),
            pltpu.VMEM((rpw // R, R), jnp.int32),
            pltpu.VMEM((rpw,), jnp.int32),
            pltpu.VMEM((rpw // R, R), jnp.int32),
            pltpu.VMEM((rpw // R, R), jnp.int32),
            pltpu.VMEM_SHARED((NS * rpw,), jnp.int32),
            pltpu.VMEM((R, DIM), jnp.float32),
            pltpu.VMEM((R, DIM), jnp.float32),
            pltpu.SemaphoreType.DMA,
            pltpu.SemaphoreType.DMA,
            pltpu.SemaphoreType.DMA,
            pltpu.SemaphoreType.DMA,
        ],
    )(x2, t1, emb)


ROW_BLOCK = 1024


def _tc_body(t_ref, x_ref, emb_ref, o_ref):
    tt = t_ref[0].reshape(t_ref.shape[2], 1)
    e0 = emb_ref[0, :][None, :]
    e1 = emb_ref[1, :][None, :]
    e2 = emb_ref[2, :][None, :]
    sel = jnp.where(tt == 0, e0, jnp.where(tt == 1, e1, e2))
    o_ref[...] = x_ref[...] + sel


def _tc_call(x2, t1, emb):
    rows, d = x2.shape
    nblk = rows // ROW_BLOCK
    t3 = t1.reshape(nblk, 1, ROW_BLOCK)
    return pl.pallas_call(
        _tc_body,
        grid=(nblk,),
        in_specs=[
            pl.BlockSpec((1, 1, ROW_BLOCK), lambda i: (i, 0, 0)),
            pl.BlockSpec((ROW_BLOCK, d), lambda i: (i, 0)),
            pl.BlockSpec((3, d), lambda i: (0, 0)),
        ],
        out_specs=pl.BlockSpec((ROW_BLOCK, d), lambda i: (i, 0)),
        out_shape=jax.ShapeDtypeStruct((rows, d), x2.dtype),
    )(t3, x2, emb)


SC_ROWS = 6144


def kernel(x, t, emb):
    b, s, d = x.shape
    rows = b * s
    x2 = x.reshape(rows, d)
    t1 = t.astype(jnp.int32).reshape(rows)
    out = _sc_call(x2, t1, emb)
    return out.reshape(b, s, d)


# speedup vs baseline: 1.7720x; 1.7720x over previous
import functools

import jax
import jax.numpy as jnp
from jax import lax
from jax.experimental import pallas as pl
from jax.experimental.pallas import tpu as pltpu
from jax.experimental.pallas import tpu_sc as plsc

DIM = 1024
NC, NS, L = 2, 16, 16
NW = NC * NS
R = 32


def _sc_body(x_hbm, t_hbm, emb_hbm, out_hbm,
             embv, tv, xb0, xb1, si0, si1, so0, so1):
    wid = lax.axis_index("s") * NC + lax.axis_index("c")
    rows_per_w = x_hbm.shape[0] // NW
    nchunks = rows_per_w // R
    base = wid * rows_per_w

    pltpu.sync_copy(emb_hbm, embv)
    pltpu.sync_copy(t_hbm.at[pl.ds(base, rows_per_w)],
                    tv.at[pl.ds(0, rows_per_w)])

    bufs, si, so = [xb0, xb1], [si0, si1], [so0, so1]
    in_d, out_d = {}, {}

    def start_in(g):
        in_d[g] = pltpu.async_copy(
            x_hbm.at[pl.ds(base + g * R, R)], bufs[g & 1], si[g & 1])

    def start_out(g):
        out_d[g] = pltpu.async_copy(
            bufs[g & 1], out_hbm.at[pl.ds(base + g * R, R)], so[g & 1])

    start_in(0)
    for g in range(nchunks):
        buf = bufs[g & 1]
        if g + 1 < nchunks:
            if g >= 1:
                out_d[g - 1].wait()
            start_in(g + 1)
        in_d[g].wait()

        g0 = g * R

        @plsc.parallel_loop(0, R, step=1, unroll=2)
        def row_body(r, buf=buf, g0=g0):
            tval = tv[pl.ds(g0 + r, L)][0]

            @plsc.parallel_loop(0, DIM, step=L, unroll=8)
            def dim_body(cc):
                buf[r, pl.ds(cc, L)] = (
                    buf[r, pl.ds(cc, L)] + embv[tval, pl.ds(cc, L)])

        start_out(g)

    out_d[nchunks - 2].wait()
    out_d[nchunks - 1].wait()


def _sc_call(x2, t1, emb):
    rows = x2.shape[0]
    return pl.kernel(
        _sc_body,
        out_type=jax.ShapeDtypeStruct((rows, DIM), jnp.float32),
        mesh=plsc.VectorSubcoreMesh(core_axis_name="c", subcore_axis_name="s"),
        scratch_types=[
            pltpu.VMEM((3, DIM), jnp.float32),
            pltpu.VMEM((rows // NW + L,), jnp.int32),
            pltpu.VMEM((R, DIM), jnp.float32),
            pltpu.VMEM((R, DIM), jnp.float32),
            pltpu.SemaphoreType.DMA,
            pltpu.SemaphoreType.DMA,
            pltpu.SemaphoreType.DMA,
            pltpu.SemaphoreType.DMA,
        ],
    )(x2, t1, emb)


def kernel(x, t, emb):
    b, s, d = x.shape
    rows = b * s
    x2 = x.reshape(rows, d)
    t1 = t.astype(jnp.int32).reshape(rows)
    out = _sc_call(x2, t1, emb)
    return out.reshape(b, s, d)
